# baseline (speedup 1.0000x reference)
import functools

import jax
import jax.numpy as jnp
from jax import lax
from jax.experimental import pallas as pl
from jax.experimental.pallas import tpu as pltpu
from jax.experimental.pallas import tpu_sc as plsc

VOCAB = 100000
D_MODEL = 2048
RANK = 32
SCALING = 16.0 / 32.0
DW = D_MODEL // 4
CW = 640

BATCH, SEQ = 1024, 50
NTOK = BATCH * SEQ
NC, NS = 2, 16
NW = NC * NS

NSPLIT = 2
SEQ_H = SEQ // NSPLIT
NTOK_H = NTOK // NSPLIT
TPW = NTOK_H // NW
CHUNK = 80
NCHUNK = TPW // CHUNK
NPAIR = NCHUNK // 2


def _sc_gather(idx_hbm, table_hbm, out_hbm, idx_v, qb0, qb1,
               semg0, semg1, semw0, semw1):
    wid = lax.axis_index("s") * NC + lax.axis_index("c")
    base = wid * TPW
    pltpu.sync_copy(idx_hbm.at[pl.ds(base, TPW)], idx_v)

    qb = (qb0, qb1)
    semg, semw = (semg0, semg1), (semw0, semw1)

    def idxc(j):
        return idx_v.at[pl.ds(j * CHUNK, CHUNK)]

    def out_sl(j):
        return out_hbm.at[pl.ds(base + j * CHUNK, CHUNK)]

    def start_gather(j, p):
        pltpu.async_copy(table_hbm.at[idxc(j)], qb[p], semg[p])

    def wait_gather(j, p):
        pltpu.make_async_copy(table_hbm.at[idxc(j)], qb[p], semg[p]).wait()

    def start_wb(j, p):
        pltpu.async_copy(qb[p], out_sl(j), semw[p])

    def wait_wb(j, p):
        pltpu.make_async_copy(qb[p], out_sl(j), semw[p]).wait()

    start_gather(0, 0)
    start_gather(1, 1)

    def body(i, carry):
        j0 = 2 * i
        for p in (0, 1):
            wait_gather(j0 + p, p)
            start_wb(j0 + p, p)
        for p in (0, 1):
            wait_wb(j0 + p, p)
            start_gather(j0 + p + 2, p)
        return carry

    lax.fori_loop(0, NPAIR - 1, body, 0)

    j0 = NCHUNK - 2
    for p in (0, 1):
        wait_gather(j0 + p, p)
        start_wb(j0 + p, p)
    for p in (0, 1):
        wait_wb(j0 + p, p)


@functools.cache
def _sc_gather_call():
    return pl.kernel(
        _sc_gather,
        mesh=plsc.VectorSubcoreMesh(core_axis_name="c", subcore_axis_name="s",
                                    num_cores=NC, num_subcores=NS),
        out_type=jax.ShapeDtypeStruct((NTOK_H, CW), jnp.int32),
        scratch_types=[
            pltpu.VMEM((TPW,), jnp.int32),
            pltpu.VMEM((CHUNK, CW), jnp.int32),
            pltpu.VMEM((CHUNK, CW), jnp.int32),
            pltpu.SemaphoreType.DMA,
            pltpu.SemaphoreType.DMA,
            pltpu.SemaphoreType.DMA,
            pltpu.SemaphoreType.DMA,
        ],
    )


PBLK = 800


def _pack_body(w_ref, a_ref, t_ref):
    wu = lax.bitcast_convert_type(w_ref[...], jnp.uint8)
    b = [wu[:, j * DW:(j + 1) * DW].astype(jnp.int32) for j in range(4)]
    t_ref[:, 0:DW] = (b[0] | lax.shift_left(b[1], 8)
                      | lax.shift_left(b[2], 16) | lax.shift_left(b[3], 24))
    t_ref[:, DW:DW + RANK] = lax.bitcast_convert_type(a_ref[...], jnp.int32)


def _tc_pack(w, a):
    return pl.pallas_call(
        _pack_body,
        grid=(VOCAB // PBLK,),
        in_specs=[pl.BlockSpec((PBLK, D_MODEL), lambda i: (i, 0)),
                  pl.BlockSpec((PBLK, RANK), lambda i: (i, 0))],
        out_specs=pl.BlockSpec((PBLK, CW), lambda i: (i, 0)),
        out_shape=jax.ShapeDtypeStruct((VOCAB, CW), jnp.int32),
    )(w, a)


BT = 1
TBLK = BT * BATCH


def _make_tc_body(half):
    def _tc_body(scale_ref, q_ref, b_ref, *rest):
        o_ref = rest[-1]
        scale = scale_ref[0]
        q = q_ref[:, 0:DW]
        a = lax.bitcast_convert_type(q_ref[:, DW:DW + RANK], jnp.float32)
        lora = jnp.dot(a, b_ref[...],
                       preferred_element_type=jnp.float32) * SCALING
        for j in range(4):
            bj = lax.shift_right_arithmetic(
                lax.shift_left(q, jnp.int32(24 - 8 * j)), jnp.int32(24))
            slab = (bj.astype(jnp.float32) * scale
                    + lora[:, j * DW:(j + 1) * DW])
            o_ref[:, :, j * DW:(j + 1) * DW] = slab.reshape(BT, BATCH, DW)
    return _tc_body


def _tc_dense(scale, qa, b, half, prev=None):
    in_specs = [
        pl.BlockSpec(memory_space=pltpu.SMEM),
        pl.BlockSpec((TBLK, CW), lambda i: (i, 0)),
        pl.BlockSpec((RANK, D_MODEL), lambda i: (0, 0)),
    ]
    args = [scale, qa, b]
    kwargs = {}
    if prev is not None:
        in_specs.append(pl.BlockSpec(memory_space=pl.ANY))
        args.append(prev)
        kwargs["input_output_aliases"] = {3: 0}
    off = half * (SEQ_H // BT)
    return pl.pallas_call(
        _make_tc_body(half),
        grid=(SEQ_H // BT,),
        in_specs=in_specs,
        out_specs=pl.BlockSpec((BT, BATCH, D_MODEL),
                               lambda i: (i + off, 0, 0)),
        out_shape=jax.ShapeDtypeStruct((SEQ, BATCH, D_MODEL), jnp.float32),
        **kwargs,
    )(*args)


def kernel(input_ids, w_base_q, w_scale, adapter_A, adapter_B):
    idx = input_ids.astype(jnp.int32).T.reshape(-1)
    t32 = _tc_pack(w_base_q, adapter_A)
    sc = _sc_gather_call()
    qa0 = sc(idx[:NTOK_H], t32)
    qa1 = sc(idx[NTOK_H:], t32)
    out = _tc_dense(w_scale, qa0, adapter_B, 0)
    out = _tc_dense(w_scale, qa1, adapter_B, 1, prev=out)
    return jnp.swapaxes(out, 0, 1)


# speedup vs baseline: 1.0161x; 1.0161x over previous
import functools

import jax
import jax.numpy as jnp
from jax import lax
from jax.experimental import pallas as pl
from jax.experimental.pallas import tpu as pltpu
from jax.experimental.pallas import tpu_sc as plsc

VOCAB = 100000
D_MODEL = 2048
RANK = 32
RPAD = 128
SCALING = 16.0 / 32.0
DW = D_MODEL // 4

NTOK = 1024 * 50
NC, NS = 2, 16
NW = NC * NS
TPW = NTOK // NW
CHUNK = 80
NCHUNK = TPW // CHUNK
NPAIR = NCHUNK // 2


def _sc_gather(idx_hbm, tableq_hbm, a_hbm, outq_hbm, outa_hbm,
               idx_v, qb0, qb1, ab0, ab1, semg0, semg1, semw0, semw1):
    wid = lax.axis_index("s") * NC + lax.axis_index("c")
    base = wid * TPW
    pltpu.sync_copy(idx_hbm.at[pl.ds(base, TPW)], idx_v)

    qb, ab = (qb0, qb1), (ab0, ab1)
    semg, semw = (semg0, semg1), (semw0, semw1)

    def idxc(j):
        return idx_v.at[pl.ds(j * CHUNK, CHUNK)]

    def out_sl(j, ref):
        return ref.at[pl.ds(base + j * CHUNK, CHUNK)]

    def start_gather(j, p):
        pltpu.async_copy(tableq_hbm.at[idxc(j)], qb[p], semg[p])
        pltpu.async_copy(a_hbm.at[idxc(j)], ab[p], semg[p])

    def wait_gather(j, p):
        pltpu.make_async_copy(tableq_hbm.at[idxc(j)], qb[p], semg[p]).wait()
        pltpu.make_async_copy(a_hbm.at[idxc(j)], ab[p], semg[p]).wait()

    def start_wb(j, p):
        pltpu.async_copy(qb[p], out_sl(j, outq_hbm), semw[p])
        pltpu.async_copy(ab[p], out_sl(j, outa_hbm), semw[p])

    def wait_wb(j, p):
        pltpu.make_async_copy(qb[p], out_sl(j, outq_hbm), semw[p]).wait()
        pltpu.make_async_copy(ab[p], out_sl(j, outa_hbm), semw[p]).wait()

    start_gather(0, 0)
    start_gather(1, 1)

    def body(i, carry):
        j0 = 2 * i
        for p in (0, 1):
            j = j0 + p
            wait_gather(j, p)
            start_wb(j, p)
        for p in (0, 1):
            j = j0 + p
            wait_wb(j, p)
            start_gather(j + 2, p)
        return carry

    lax.fori_loop(0, NPAIR - 1, body, 0)

    j0 = NCHUNK - 2
    for p in (0, 1):
        wait_gather(j0 + p, p)
        start_wb(j0 + p, p)
    for p in (0, 1):
        wait_wb(j0 + p, p)


@functools.cache
def _sc_gather_call():
    return pl.kernel(
        _sc_gather,
        mesh=plsc.VectorSubcoreMesh(core_axis_name="c", subcore_axis_name="s",
                                    num_cores=NC, num_subcores=NS),
        out_type=(
            jax.ShapeDtypeStruct((NTOK, DW), jnp.int32),
            jax.ShapeDtypeStruct((NTOK, RPAD), jnp.float32),
        ),
        scratch_types=[
            pltpu.VMEM((TPW,), jnp.int32),
            pltpu.VMEM((CHUNK, DW), jnp.int32),
            pltpu.VMEM((CHUNK, DW), jnp.int32),
            pltpu.VMEM((CHUNK, RPAD), jnp.float32),
            pltpu.VMEM((CHUNK, RPAD), jnp.float32),
            pltpu.SemaphoreType.DMA,
            pltpu.SemaphoreType.DMA,
            pltpu.SemaphoreType.DMA,
            pltpu.SemaphoreType.DMA,
        ],
    )


PBLK = 800


def _pack_body(w_ref, t_ref):
    b = [(w_ref[:, j * DW:(j + 1) * DW].astype(jnp.int32) & 0xFF)
         for j in range(4)]
    t_ref[...] = (b[0] | lax.shift_left(b[1], 8)
                  | lax.shift_left(b[2], 16) | lax.shift_left(b[3], 24))


def _tc_pack(w):
    return pl.pallas_call(
        _pack_body,
        grid=(VOCAB // PBLK,),
        in_specs=[pl.BlockSpec((PBLK, D_MODEL), lambda i: (i, 0))],
        out_specs=pl.BlockSpec((PBLK, DW), lambda i: (i, 0)),
        out_shape=jax.ShapeDtypeStruct((VOCAB, DW), jnp.int32),
    )(w)


BATCH, SEQ = 1024, 50
BT = 1
TBLK = BT * BATCH


def _tc_body(scale_ref, q_ref, a_ref, b_ref, o_ref):
    scale = scale_ref[0]
    q = q_ref[...]
    lora = jnp.dot(a_ref[...], b_ref[...],
                   preferred_element_type=jnp.float32) * SCALING
    for j in range(4):
        bj = lax.shift_right_arithmetic(
            lax.shift_left(q, jnp.int32(24 - 8 * j)), jnp.int32(24))
        slab = bj.astype(jnp.float32) * scale + lora[:, j * DW:(j + 1) * DW]
        o_ref[:, :, j * DW:(j + 1) * DW] = slab.reshape(BT, BATCH, DW)


def _tc_dense(scale, q, a, b):
    grid = (SEQ // BT,)
    return pl.pallas_call(
        _tc_body,
        grid=grid,
        in_specs=[
            pl.BlockSpec(memory_space=pltpu.SMEM),
            pl.BlockSpec((TBLK, DW), lambda i: (i, 0)),
            pl.BlockSpec((TBLK, RPAD), lambda i: (i, 0)),
            pl.BlockSpec((RPAD, D_MODEL), lambda i: (0, 0)),
        ],
        out_specs=pl.BlockSpec((BT, BATCH, D_MODEL), lambda i: (i, 0, 0)),
        out_shape=jax.ShapeDtypeStruct((SEQ, BATCH, D_MODEL), jnp.float32),
    )(scale, q, a, b)


def kernel(input_ids, w_base_q, w_scale, adapter_A, adapter_B):
    idx = input_ids.astype(jnp.int32).T.reshape(-1)
    t32 = _tc_pack(w_base_q)
    a_pad = jnp.pad(adapter_A, ((0, 0), (0, RPAD - RANK)))
    b_pad = jnp.pad(adapter_B, ((0, RPAD - RANK), (0, 0)))
    q_rows, a_rows = _sc_gather_call()(idx, t32, a_pad)
    out_tm = _tc_dense(w_scale, q_rows, a_rows, b_pad)
    return jnp.swapaxes(out_tm, 0, 1)
